# initial kernel scaffold (unmeasured)
import jax
import jax.numpy as jnp
from jax import lax
from jax.experimental import pallas as pl
from jax.experimental.pallas import tpu as pltpu

N_DEV = 4
T = 2048
T_BLK = T // 2
V_SHARD = 8192
V = 2 * V_SHARD
ROW_CHUNK = 128

_BLOCK_SLOTS = {0: (0, 1), 1: (3, 2)}


def _ring_coords(p):
    px = p // 2
    py = (p % 2) ^ px
    return px, py


def _gather_softmax(tile):
    def body(tile_hbm, out_ref, comm_ref, probs_ref, in_sem,
             send_sems, recv_sems, copy_sems):
        my_x = lax.axis_index("x")
        my_y = lax.axis_index("y")
        pos = 2 * my_x + (my_y ^ my_x)
        rx, ry = _ring_coords((pos + 1) % N_DEV)
        lx, ly = _ring_coords((pos + 3) % N_DEV)

        in_copy = pltpu.make_async_copy(tile_hbm, comm_ref.at[pos], in_sem)
        in_copy.start()

        barrier = pltpu.get_barrier_semaphore()
        for nx, ny in ((lx, ly), (rx, ry)):
            pl.semaphore_signal(barrier, inc=1, device_id=(nx, ny),
                                device_id_type=pl.DeviceIdType.MESH)
        pl.semaphore_wait(barrier, 2)
        in_copy.wait()

        for h in range(N_DEV - 1):
            origin = (pos - h) % N_DEV
            rdma = pltpu.make_async_remote_copy(
                src_ref=comm_ref.at[origin],
                dst_ref=comm_ref.at[origin],
                send_sem=send_sems.at[h],
                recv_sem=recv_sems.at[h],
                device_id=(rx, ry),
                device_id_type=pl.DeviceIdType.MESH,
            )
            rdma.start()
            rdma.wait()

        n_chunks = T_BLK // ROW_CHUNK
        prev = None
        for b in (0, 1):
            s0, s1 = _BLOCK_SLOTS[b]
            for c in range(n_chunks):
                rows = pl.ds(c * ROW_CHUNK, ROW_CHUNK)
                l0 = comm_ref[s0, rows, :].astype(jnp.float32)
                l1 = comm_ref[s1, rows, :].astype(jnp.float32)
                m = jnp.maximum(l0.max(-1, keepdims=True),
                                l1.max(-1, keepdims=True))
                e0 = jnp.exp(l0 - m)
                e1 = jnp.exp(l1 - m)
                r = 1.0 / (e0.sum(-1, keepdims=True)
                           + e1.sum(-1, keepdims=True))
                if prev is not None:
                    prev[0].wait()
                    prev[1].wait()
                probs_ref[0, :, :] = e0 * r
                probs_ref[1, :, :] = e1 * r
                row0 = b * T_BLK + c * ROW_CHUNK
                cp0 = pltpu.make_async_copy(
                    probs_ref.at[0],
                    out_ref.at[pl.ds(row0, ROW_CHUNK), pl.ds(0, V_SHARD)],
                    copy_sems.at[0])
                cp1 = pltpu.make_async_copy(
                    probs_ref.at[1],
                    out_ref.at[pl.ds(row0, ROW_CHUNK), pl.ds(V_SHARD, V_SHARD)],
                    copy_sems.at[1])
                cp0.start()
                cp1.start()
                prev = (cp0, cp1)
        prev[0].wait()
        prev[1].wait()

    return pl.pallas_call(
        body,
        out_shape=jax.ShapeDtypeStruct((T, V), jnp.float32),
        in_specs=[pl.BlockSpec(memory_space=pltpu.ANY)],
        out_specs=pl.BlockSpec(memory_space=pltpu.ANY),
        scratch_shapes=[
            pltpu.VMEM((N_DEV, T_BLK, V_SHARD), jnp.bfloat16),
            pltpu.VMEM((2, ROW_CHUNK, V_SHARD), jnp.float32),
            pltpu.SemaphoreType.DMA,
            pltpu.SemaphoreType.DMA((N_DEV - 1,)),
            pltpu.SemaphoreType.DMA((N_DEV - 1,)),
            pltpu.SemaphoreType.DMA((2,)),
        ],
        compiler_params=pltpu.CompilerParams(collective_id=0),
    )(tile)


def kernel(x, W):
    my_x = lax.axis_index("x")
    xb = x.astype(jnp.bfloat16)
    Wb = W.astype(jnp.bfloat16)
    x_rows = lax.dynamic_slice_in_dim(xb, my_x * T_BLK, T_BLK, axis=0)
    tile = jnp.dot(x_rows, Wb, preferred_element_type=jnp.float32)
    return _gather_softmax(tile.astype(jnp.bfloat16))


# baseline (device time: 840643 ns/iter reference)
import jax
import jax.numpy as jnp
from jax import lax
from jax.experimental import pallas as pl
from jax.experimental.pallas import tpu as pltpu

N_DEV = 4
T = 2048
T_BLK = T // 2
V_SHARD = 8192
V = 2 * V_SHARD
ROW_CHUNK = 128

_BLOCK_SLOTS = {0: (0, 1), 1: (3, 2)}


def _ring_coords(p):
    px = p // 2
    py = (p % 2) ^ px
    return px, py


def _gather_softmax(tile):
    def body(tile_hbm, out_ref, comm_ref, stage_ref, probs_ref, in_sem,
             send_sems, recv_sems, stage_sems, copy_sems):
        my_x = lax.axis_index("x")
        my_y = lax.axis_index("y")
        pos = 2 * my_x + (my_y ^ my_x)
        rx, ry = _ring_coords((pos + 1) % N_DEV)
        lx, ly = _ring_coords((pos + 3) % N_DEV)

        in_copy = pltpu.make_async_copy(tile_hbm, comm_ref.at[pos], in_sem)
        in_copy.start()

        barrier = pltpu.get_barrier_semaphore()
        for nx, ny in ((lx, ly), (rx, ry)):
            pl.semaphore_signal(barrier, inc=1, device_id=(nx, ny),
                                device_id_type=pl.DeviceIdType.MESH)
        pl.semaphore_wait(barrier, 2)
        in_copy.wait()

        for h in range(N_DEV - 1):
            origin = (pos - h) % N_DEV
            rdma = pltpu.make_async_remote_copy(
                src_ref=comm_ref.at[origin],
                dst_ref=comm_ref.at[origin],
                send_sem=send_sems.at[h],
                recv_sem=recv_sems.at[h],
                device_id=(rx, ry),
                device_id_type=pl.DeviceIdType.MESH,
            )
            rdma.start()
            rdma.wait()

        n_chunks = T_BLK // ROW_CHUNK
        prev = None
        for b in (0, 1):
            s0, s1 = _BLOCK_SLOTS[b]
            for c in range(n_chunks):
                rows = pl.ds(c * ROW_CHUNK, ROW_CHUNK)
                ld0 = pltpu.make_async_copy(
                    comm_ref.at[s0, rows, :], stage_ref.at[0],
                    stage_sems.at[0])
                ld1 = pltpu.make_async_copy(
                    comm_ref.at[s1, rows, :], stage_ref.at[1],
                    stage_sems.at[1])
                ld0.start()
                ld1.start()
                ld0.wait()
                ld1.wait()
                l0 = stage_ref[0].astype(jnp.float32)
                l1 = stage_ref[1].astype(jnp.float32)
                m = jnp.maximum(l0.max(-1, keepdims=True),
                                l1.max(-1, keepdims=True))
                e0 = jnp.exp(l0 - m)
                e1 = jnp.exp(l1 - m)
                r = 1.0 / (e0.sum(-1, keepdims=True)
                           + e1.sum(-1, keepdims=True))
                if prev is not None:
                    prev[0].wait()
                    prev[1].wait()
                probs_ref[0, :, :] = e0 * r
                probs_ref[1, :, :] = e1 * r
                row0 = b * T_BLK + c * ROW_CHUNK
                cp0 = pltpu.make_async_copy(
                    probs_ref.at[0],
                    out_ref.at[pl.ds(row0, ROW_CHUNK), pl.ds(0, V_SHARD)],
                    copy_sems.at[0])
                cp1 = pltpu.make_async_copy(
                    probs_ref.at[1],
                    out_ref.at[pl.ds(row0, ROW_CHUNK), pl.ds(V_SHARD, V_SHARD)],
                    copy_sems.at[1])
                cp0.start()
                cp1.start()
                prev = (cp0, cp1)
        prev[0].wait()
        prev[1].wait()

    def reorder(tile_hbm, out_ref, comm_ref, *scratch):
        return body(tile_hbm, out_ref, comm_ref, *scratch)

    out, _ = pl.pallas_call(
        reorder,
        out_shape=[
            jax.ShapeDtypeStruct((T, V), jnp.float32),
            jax.ShapeDtypeStruct((N_DEV, T_BLK, V_SHARD), jnp.bfloat16),
        ],
        in_specs=[pl.BlockSpec(memory_space=pl.ANY)],
        out_specs=[
            pl.BlockSpec(memory_space=pl.ANY),
            pl.BlockSpec(memory_space=pl.ANY),
        ],
        scratch_shapes=[
            pltpu.VMEM((2, ROW_CHUNK, V_SHARD), jnp.bfloat16),
            pltpu.VMEM((2, ROW_CHUNK, V_SHARD), jnp.float32),
            pltpu.SemaphoreType.DMA,
            pltpu.SemaphoreType.DMA((N_DEV - 1,)),
            pltpu.SemaphoreType.DMA((N_DEV - 1,)),
            pltpu.SemaphoreType.DMA((2,)),
            pltpu.SemaphoreType.DMA((2,)),
        ],
        compiler_params=pltpu.CompilerParams(collective_id=0),
    )(tile)
    return out


def kernel(x, W):
    my_x = lax.axis_index("x")
    xb = x.astype(jnp.bfloat16)
    Wb = W.astype(jnp.bfloat16)
    x_rows = lax.dynamic_slice_in_dim(xb, my_x * T_BLK, T_BLK, axis=0)
    tile = jnp.dot(x_rows, Wb, preferred_element_type=jnp.float32)
    return _gather_softmax(tile.astype(jnp.bfloat16))


# device time: 548457 ns/iter; 1.5327x vs baseline; 1.5327x over previous
import jax
import jax.numpy as jnp
from jax import lax
from jax.experimental import pallas as pl
from jax.experimental.pallas import tpu as pltpu

N_DEV = 4
T = 2048
T_BLK = T // 2
V_SHARD = 8192
V = 2 * V_SHARD
N_CHUNK = 8
RC = T_BLK // N_CHUNK


def _gather_softmax(tile):
    def body(tile_hbm, out_ref, comm_ref, stage_ref, probs_ref,
             sr_own, rl_own, sl_own, rr_own, sr_fwd, rl_fwd,
             stage_sems, copy_sems):
        my_x = lax.axis_index("x")
        my_y = lax.axis_index("y")
        pos = 2 * my_x + (my_y ^ my_x)
        lpos = (pos + 3) % N_DEV
        rpos = (pos + 1) % N_DEV
        opp = (pos + 2) % N_DEV

        def coords(p):
            return p // 2, (p % 2) ^ (p // 2)

        lx, ly = coords(lpos)
        rx, ry = coords(rpos)

        barrier = pltpu.get_barrier_semaphore()
        for nx, ny in ((lx, ly), (rx, ry)):
            pl.semaphore_signal(barrier, inc=1, device_id=(nx, ny),
                                device_id_type=pl.DeviceIdType.MESH)
        pl.semaphore_wait(barrier, 2)

        def rows(c):
            return pl.ds(c * RC, RC)

        d_ro = [pltpu.make_async_remote_copy(
            src_ref=tile_hbm.at[rows(c), :],
            dst_ref=comm_ref.at[pos, rows(c), :],
            send_sem=sr_own.at[c], recv_sem=rl_own.at[c],
            device_id=(rx, ry), device_id_type=pl.DeviceIdType.MESH,
        ) for c in range(N_CHUNK)]
        d_lo = [pltpu.make_async_remote_copy(
            src_ref=tile_hbm.at[rows(c), :],
            dst_ref=comm_ref.at[pos, rows(c), :],
            send_sem=sl_own.at[c], recv_sem=rr_own.at[c],
            device_id=(lx, ly), device_id_type=pl.DeviceIdType.MESH,
        ) for c in range(N_CHUNK)]
        d_fwd = [pltpu.make_async_remote_copy(
            src_ref=comm_ref.at[lpos, rows(c), :],
            dst_ref=comm_ref.at[lpos, rows(c), :],
            send_sem=sr_fwd.at[c], recv_sem=rl_fwd.at[c],
            device_id=(rx, ry), device_id_type=pl.DeviceIdType.MESH,
        ) for c in range(N_CHUNK)]

        for c in range(N_CHUNK):
            d_ro[c].start()
            d_lo[c].start()

        prev_cp = []

        def chunk_softmax(src_a, col_a, src_b, col_b, row0):
            nonlocal prev_cp
            lda = pltpu.make_async_copy(src_a, stage_ref.at[0],
                                        stage_sems.at[0])
            ldb = pltpu.make_async_copy(src_b, stage_ref.at[1],
                                        stage_sems.at[1])
            lda.start()
            ldb.start()
            lda.wait()
            ldb.wait()
            la = stage_ref[0].astype(jnp.float32)
            lb = stage_ref[1].astype(jnp.float32)
            m = jnp.maximum(la.max(-1, keepdims=True),
                            lb.max(-1, keepdims=True))
            ea = jnp.exp(la - m)
            eb = jnp.exp(lb - m)
            r = 1.0 / (ea.sum(-1, keepdims=True) + eb.sum(-1, keepdims=True))
            for cp in prev_cp:
                cp.wait()
            probs_ref[0, :, :] = ea * r
            probs_ref[1, :, :] = eb * r
            cpa = pltpu.make_async_copy(
                probs_ref.at[0],
                out_ref.at[pl.ds(row0, RC), pl.ds(col_a * V_SHARD, V_SHARD)],
                copy_sems.at[0])
            cpb = pltpu.make_async_copy(
                probs_ref.at[1],
                out_ref.at[pl.ds(row0, RC), pl.ds(col_b * V_SHARD, V_SHARD)],
                copy_sems.at[1])
            cpa.start()
            cpb.start()
            prev_cp = [cpa, cpb]

        for c in range(N_CHUNK):
            d_ro[c].wait_recv()
            d_fwd[c].start()
            d_lo[c].wait_recv()
            chunk_softmax(tile_hbm.at[rows(c), :], my_y,
                          comm_ref.at[pos ^ 1, rows(c), :], 1 - my_y,
                          my_x * T_BLK + c * RC)

        for c in range(N_CHUNK):
            d_fwd[c].wait_recv()
            chunk_softmax(comm_ref.at[opp ^ 1, rows(c), :], my_y,
                          comm_ref.at[opp, rows(c), :], 1 - my_y,
                          (1 - my_x) * T_BLK + c * RC)

        for cp in prev_cp:
            cp.wait()
        for c in range(N_CHUNK):
            d_ro[c].wait_send()
            d_lo[c].wait_send()
            d_fwd[c].wait_send()

    out, _ = pl.pallas_call(
        body,
        out_shape=[
            jax.ShapeDtypeStruct((T, V), jnp.float32),
            jax.ShapeDtypeStruct((N_DEV, T_BLK, V_SHARD), jnp.bfloat16),
        ],
        in_specs=[pl.BlockSpec(memory_space=pl.ANY)],
        out_specs=[
            pl.BlockSpec(memory_space=pl.ANY),
            pl.BlockSpec(memory_space=pl.ANY),
        ],
        scratch_shapes=[
            pltpu.VMEM((2, RC, V_SHARD), jnp.bfloat16),
            pltpu.VMEM((2, RC, V_SHARD), jnp.float32),
            pltpu.SemaphoreType.DMA((N_CHUNK,)),
            pltpu.SemaphoreType.DMA((N_CHUNK,)),
            pltpu.SemaphoreType.DMA((N_CHUNK,)),
            pltpu.SemaphoreType.DMA((N_CHUNK,)),
            pltpu.SemaphoreType.DMA((N_CHUNK,)),
            pltpu.SemaphoreType.DMA((N_CHUNK,)),
            pltpu.SemaphoreType.DMA((2,)),
            pltpu.SemaphoreType.DMA((2,)),
        ],
        compiler_params=pltpu.CompilerParams(collective_id=0),
    )(tile)
    return out


def kernel(x, W):
    my_x = lax.axis_index("x")
    xb = x.astype(jnp.bfloat16)
    Wb = W.astype(jnp.bfloat16)
    x_rows = lax.dynamic_slice_in_dim(xb, my_x * T_BLK, T_BLK, axis=0)
    tile = jnp.dot(x_rows, Wb, preferred_element_type=jnp.float32)
    return _gather_softmax(tile.astype(jnp.bfloat16))
